# Initial kernel scaffold; baseline (speedup 1.0000x reference)
import jax
import jax.numpy as jnp
from jax.experimental import pallas as pl


def kernel(x, tables):
    raise NotImplementedError("write your pallas kernel here")


# speedup vs baseline: 17.7529x; 17.7529x over previous
import functools

import jax
import jax.numpy as jnp
import numpy as np
from jax import lax
from jax.experimental import pallas as pl
from jax.experimental.pallas import tpu as pltpu
from jax.experimental.pallas import tpu_sc as plsc

NUM_LEVELS = 16
MIN_RES = 16.0
MAX_RES = 512.0
TABLE_SZ = 524288
MASK = TABLE_SZ - 1
FEATURE_DIM = 2
BATCH = 262144
C1 = np.int32(-1640531535)
C2 = np.int32(805459861)

NC, NS, L = 2, 16, 16
NW = NC * NS
SAMP = BATCH // NW
N = 1024
NCHUNK = SAMP // N
G = N // L


def _body(x0_hbm, x1_hbm, x2_hbm, tab_hbm, res_hbm, out_hbm,
          x0_v, x1_v, x2_v, res_v, idx_v, rows_v, out_v, sem):
    wid = lax.axis_index("s") * NC + lax.axis_index("c")
    base = wid * SAMP

    pltpu.sync_copy(res_hbm, res_v)

    iota = lax.iota(jnp.int32, L)

    @pl.loop(0, NCHUNK)
    def _chunk(c):
        gbase = base + c * N
        pltpu.sync_copy(x0_hbm.at[pl.ds(gbase, N)], x0_v)
        pltpu.sync_copy(x1_hbm.at[pl.ds(gbase, N)], x1_v)
        pltpu.sync_copy(x2_hbm.at[pl.ds(gbase, N)], x2_v)

        for level in range(NUM_LEVELS):
            res = res_v[level]
            lofs = jnp.full((L,), level * TABLE_SZ, jnp.int32)

            @pl.loop(0, G)
            def _hash(g):
                s0 = g * L
                xv = x0_v[pl.ds(s0, L)]
                yv = x1_v[pl.ds(s0, L)]
                zv = x2_v[pl.ds(s0, L)]
                sx = res * xv
                sy = res * yv
                sz = res * zv
                xl = sx.astype(jnp.int32)
                yl = sy.astype(jnp.int32)
                zl = sz.astype(jnp.int32)
                xh = jnp.where(sx > xl.astype(jnp.float32), xl + 1, xl)
                yh = jnp.where(sy > yl.astype(jnp.float32), yl + 1, yl)
                zh = jnp.where(sz > zl.astype(jnp.float32), zl + 1, zl)
                cyl = C1 * yl
                cyh = C1 * yh
                czl = C2 * zl
                czh = C2 * zh
                a = cyl ^ czl
                b = cyh ^ czl
                cc = cyl ^ czh
                d = cyh ^ czh
                hs = (xl ^ a, xh ^ a, xh ^ b, xl ^ b,
                      xl ^ cc, xh ^ cc, xh ^ d, xl ^ d)
                for k in range(8):
                    idx_v[pl.ds(k * N + s0, L)] = (hs[k] & MASK) + lofs

            ndma = (8 * N) // 128
            copies = [
                pltpu.async_copy(
                    tab_hbm.at[idx_v.at[pl.ds(j * 128, 128)]],
                    rows_v.at[pl.ds(j * 128, 128)], sem)
                for j in range(ndma)
            ]
            for cp in copies:
                cp.wait()

            @pl.loop(0, G)
            def _interp(g):
                s0 = g * L
                xv = x0_v[pl.ds(s0, L)]
                yv = x1_v[pl.ds(s0, L)]
                zv = x2_v[pl.ds(s0, L)]
                xw = xv - xv.astype(jnp.int32).astype(jnp.float32)
                yw = yv - yv.astype(jnp.int32).astype(jnp.float32)
                zw = zv - zv.astype(jnp.int32).astype(jnp.float32)
                xw1 = 1.0 - xw
                yw1 = 1.0 - yw
                zw1 = 1.0 - zw
                srow = s0 + iota
                for f in range(FEATURE_DIM):
                    col = jnp.full((L,), f, jnp.int32)
                    v = [plsc.load_gather(rows_v, [k * N + srow, col])
                         for k in range(8)]
                    c00 = v[0] * xw1 + v[1] * xw
                    c01 = v[4] * xw1 + v[5] * xw
                    c10 = v[3] * xw1 + v[2] * xw
                    c11 = v[7] * xw1 + v[6] * xw
                    c0 = c00 * yw1 + c10 * yw
                    c1 = c01 * yw1 + c11 * yw
                    outval = c0 * zw1 + c1 * zw
                    ocol = jnp.full((L,), 2 * level + f, jnp.int32)
                    plsc.store_scatter(out_v, [srow, ocol], outval)

        pltpu.sync_copy(out_v, out_hbm.at[pl.ds(gbase, N)])


@jax.jit
def kernel(x, tables):
    b = jnp.exp((jnp.log(MAX_RES) - jnp.log(MIN_RES)) / (NUM_LEVELS - 1))
    res_list = [jnp.floor(MIN_RES * b ** level) for level in range(NUM_LEVELS)]
    res_bcast = jnp.broadcast_to(
        jnp.stack(res_list).reshape(NUM_LEVELS, 1), (NUM_LEVELS, L))

    x0 = x[:, 0]
    x1 = x[:, 1]
    x2 = x[:, 2]
    tab2d = tables.reshape(NUM_LEVELS * TABLE_SZ, FEATURE_DIM)

    run = pl.kernel(
        _body,
        out_type=jax.ShapeDtypeStruct((BATCH, 2 * NUM_LEVELS), jnp.float32),
        mesh=plsc.VectorSubcoreMesh(core_axis_name="c", subcore_axis_name="s"),
        scratch_types=[
            pltpu.VMEM((N,), jnp.float32),
            pltpu.VMEM((N,), jnp.float32),
            pltpu.VMEM((N,), jnp.float32),
            pltpu.VMEM((NUM_LEVELS, L), jnp.float32),
            pltpu.VMEM((8 * N,), jnp.int32),
            pltpu.VMEM((8 * N, FEATURE_DIM), jnp.float32),
            pltpu.VMEM((N, 2 * NUM_LEVELS), jnp.float32),
            pltpu.SemaphoreType.DMA,
        ],
        compiler_params=pltpu.CompilerParams(
            needs_layout_passes=False, use_tc_tiling_on_sc=False),
    )
    return run(x0, x1, x2, tab2d, res_bcast)


# speedup vs baseline: 89.8830x; 5.0630x over previous
import jax
import jax.numpy as jnp
import numpy as np
from jax import lax
from jax.experimental import pallas as pl
from jax.experimental.pallas import tpu as pltpu
from jax.experimental.pallas import tpu_sc as plsc

NUM_LEVELS = 16
MIN_RES = 16.0
MAX_RES = 512.0
TABLE_SZ = 524288
MASK = TABLE_SZ - 1
FEATURE_DIM = 2
BATCH = 262144
C1 = np.int32(-1640531535)
C2 = np.int32(805459861)

NC, NS, L = 2, 16, 16
NW = NC * NS
SAMP = BATCH // NW
N = 1024
NCHUNK = SAMP // N
G = N // L
NIDX = 16 * N
IPD = 128
NDMA = NIDX // IPD
OUT_DIM = 2 * NUM_LEVELS
TBLOCK = 1 << 20


def _body(x0_hbm, x1_hbm, x2_hbm, tab_hbm, res_hbm, out_hbm,
          x0_v, x1_v, x2_v, res_v, idx_v, rows_v, out_v, sem):
    wid = lax.axis_index("s") * NC + lax.axis_index("c")
    base = wid * SAMP

    pltpu.sync_copy(res_hbm, res_v)

    @pl.loop(0, NCHUNK)
    def _chunk(c):
        gbase = base + c * N
        pltpu.sync_copy(x0_hbm.at[pl.ds(gbase, N)], x0_v)
        pltpu.sync_copy(x1_hbm.at[pl.ds(gbase, N)], x1_v)
        pltpu.sync_copy(x2_hbm.at[pl.ds(gbase, N)], x2_v)

        @pl.loop(0, NUM_LEVELS)
        def _level(level):
            res = res_v[pl.ds(level * L, L)]
            lbase = level * jnp.full((L,), TBLOCK, jnp.int32)

            @pl.loop(0, G)
            def _hash(g):
                s0 = g * L
                xv = x0_v[pl.ds(s0, L)]
                yv = x1_v[pl.ds(s0, L)]
                zv = x2_v[pl.ds(s0, L)]
                sx = res * xv
                sy = res * yv
                sz = res * zv
                xl = sx.astype(jnp.int32)
                yl = sy.astype(jnp.int32)
                zl = sz.astype(jnp.int32)
                xh = jnp.where(sx > xl.astype(jnp.float32), xl + 1, xl)
                yh = jnp.where(sy > yl.astype(jnp.float32), yl + 1, yl)
                zh = jnp.where(sz > zl.astype(jnp.float32), zl + 1, zl)
                cyl = C1 * yl
                cyh = C1 * yh
                czl = C2 * zl
                czh = C2 * zh
                a = cyl ^ czl
                b = cyh ^ czl
                cc = cyl ^ czh
                d = cyh ^ czh
                hs = (xl ^ a, xh ^ a, xh ^ b, xl ^ b,
                      xl ^ cc, xh ^ cc, xh ^ d, xl ^ d)
                for k in range(8):
                    v = hs[k] & MASK
                    e0 = lbase + ((v >> 7) << 8) + (v & 127)
                    idx_v[pl.ds(k * N + s0, L)] = e0
                    idx_v[pl.ds(8 * N + k * N + s0, L)] = e0 + 128

            @pl.loop(0, NDMA)
            def _fire(j):
                pltpu.async_copy(
                    tab_hbm.at[idx_v.at[pl.ds(j * IPD, IPD)]],
                    rows_v.at[pl.ds(j * IPD, IPD)], sem)

            @pl.loop(0, NDMA)
            def _drain(j):
                pltpu.make_async_copy(
                    tab_hbm.at[idx_v.at[pl.ds(0, IPD)]],
                    rows_v.at[pl.ds(0, IPD)], sem).wait()

            @pl.loop(0, G)
            def _interp(g):
                s0 = g * L
                xv = x0_v[pl.ds(s0, L)]
                yv = x1_v[pl.ds(s0, L)]
                zv = x2_v[pl.ds(s0, L)]
                xw = xv - xv.astype(jnp.int32).astype(jnp.float32)
                yw = yv - yv.astype(jnp.int32).astype(jnp.float32)
                zw = zv - zv.astype(jnp.int32).astype(jnp.float32)
                xw1 = 1.0 - xw
                yw1 = 1.0 - yw
                zw1 = 1.0 - zw
                for f in range(FEATURE_DIM):
                    fb = f * 8 * N
                    v = [rows_v[pl.ds(fb + k * N + s0, L)] for k in range(8)]
                    c00 = v[0] * xw1 + v[1] * xw
                    c01 = v[4] * xw1 + v[5] * xw
                    c10 = v[3] * xw1 + v[2] * xw
                    c11 = v[7] * xw1 + v[6] * xw
                    c0 = c00 * yw1 + c10 * yw
                    c1 = c01 * yw1 + c11 * yw
                    outval = c0 * zw1 + c1 * zw
                    fcol = 2 * level + f
                    off = ((fcol // 8) * (8 * N) + ((s0 >> 7) << 10)
                           + (fcol % 8) * 128 + (s0 & 127))
                    out_v[pl.ds(off, L)] = outval

        for r in range(4):
            pltpu.sync_copy(
                out_v.at[pl.ds(r * 8 * N, 8 * N)],
                out_hbm.at[pl.ds(r * (BATCH * 8) + gbase * 8, 8 * N)])


@jax.jit
def kernel(x, tables):
    b = jnp.exp((jnp.log(MAX_RES) - jnp.log(MIN_RES)) / (NUM_LEVELS - 1))
    res_list = [jnp.floor(MIN_RES * b ** level) for level in range(NUM_LEVELS)]
    res_bcast = jnp.broadcast_to(
        jnp.stack(res_list).reshape(NUM_LEVELS, 1), (NUM_LEVELS, L)).reshape(-1)

    x0 = x[:, 0]
    x1 = x[:, 1]
    x2 = x[:, 2]
    tab_flat = tables.reshape(NUM_LEVELS, TABLE_SZ // 128, 128, FEATURE_DIM)
    tab_flat = tab_flat.transpose(0, 1, 3, 2).reshape(-1)

    run = pl.kernel(
        _body,
        out_type=jax.ShapeDtypeStruct((BATCH * OUT_DIM,), jnp.float32),
        mesh=plsc.VectorSubcoreMesh(core_axis_name="c", subcore_axis_name="s"),
        scratch_types=[
            pltpu.VMEM((N,), jnp.float32),
            pltpu.VMEM((N,), jnp.float32),
            pltpu.VMEM((N,), jnp.float32),
            pltpu.VMEM((NUM_LEVELS * L,), jnp.float32),
            pltpu.VMEM((NIDX,), jnp.int32),
            pltpu.VMEM((NIDX,), jnp.float32),
            pltpu.VMEM((N * OUT_DIM,), jnp.float32),
            pltpu.SemaphoreType.DMA,
        ],
        compiler_params=pltpu.CompilerParams(
            needs_layout_passes=False, use_tc_tiling_on_sc=False),
    )
    out_flat = run(x0, x1, x2, tab_flat, res_bcast)
    out4 = out_flat.reshape(4, BATCH // 128, 8, 128)
    return out4.transpose(1, 3, 0, 2).reshape(BATCH, OUT_DIM)


# speedup vs baseline: 90.1914x; 1.0034x over previous
import jax
import jax.numpy as jnp
import numpy as np
from jax import lax
from jax.experimental import pallas as pl
from jax.experimental.pallas import tpu as pltpu
from jax.experimental.pallas import tpu_sc as plsc

NUM_LEVELS = 16
MIN_RES = 16.0
MAX_RES = 512.0
TABLE_SZ = 524288
MASK = TABLE_SZ - 1
FEATURE_DIM = 2
BATCH = 262144
C1 = np.int32(-1640531535)
C2 = np.int32(805459861)

NC, NS, L = 2, 16, 16
NW = NC * NS
SAMP = BATCH // NW
N = 1024
NCHUNK = SAMP // N
G = N // L
NIDX = 16 * N
IPD = 256
NDMA = NIDX // IPD
OUT_DIM = 2 * NUM_LEVELS
TBLOCK = 1 << 20


def _body(x0_hbm, x1_hbm, x2_hbm, tab_hbm, res_hbm, out_hbm,
          x0_v, x1_v, x2_v, res_v, idx_v, rows_v, out_v, sem):
    wid = lax.axis_index("s") * NC + lax.axis_index("c")
    base = wid * SAMP

    pltpu.sync_copy(res_hbm, res_v)

    @pl.loop(0, NCHUNK)
    def _chunk(c):
        gbase = base + c * N
        pltpu.sync_copy(x0_hbm.at[pl.ds(gbase, N)], x0_v)
        pltpu.sync_copy(x1_hbm.at[pl.ds(gbase, N)], x1_v)
        pltpu.sync_copy(x2_hbm.at[pl.ds(gbase, N)], x2_v)

        @pl.loop(0, NUM_LEVELS)
        def _level(level):
            res = res_v[pl.ds(level * L, L)]
            lbase = level * jnp.full((L,), TBLOCK, jnp.int32)

            @pl.loop(0, G)
            def _hash(g):
                s0 = g * L
                xv = x0_v[pl.ds(s0, L)]
                yv = x1_v[pl.ds(s0, L)]
                zv = x2_v[pl.ds(s0, L)]
                sx = res * xv
                sy = res * yv
                sz = res * zv
                xl = sx.astype(jnp.int32)
                yl = sy.astype(jnp.int32)
                zl = sz.astype(jnp.int32)
                xh = jnp.where(sx > xl.astype(jnp.float32), xl + 1, xl)
                yh = jnp.where(sy > yl.astype(jnp.float32), yl + 1, yl)
                zh = jnp.where(sz > zl.astype(jnp.float32), zl + 1, zl)
                cyl = C1 * yl
                cyh = C1 * yh
                czl = C2 * zl
                czh = C2 * zh
                a = cyl ^ czl
                b = cyh ^ czl
                cc = cyl ^ czh
                d = cyh ^ czh
                hs = (xl ^ a, xh ^ a, xh ^ b, xl ^ b,
                      xl ^ cc, xh ^ cc, xh ^ d, xl ^ d)
                for k in range(8):
                    v = hs[k] & MASK
                    e0 = lbase + ((v >> 7) << 8) + (v & 127)
                    idx_v[pl.ds(k * N + s0, L)] = e0
                    idx_v[pl.ds(8 * N + k * N + s0, L)] = e0 + 128

            @pl.loop(0, NDMA)
            def _fire(j):
                pltpu.async_copy(
                    tab_hbm.at[idx_v.at[pl.ds(j * IPD, IPD)]],
                    rows_v.at[pl.ds(j * IPD, IPD)], sem)

            @pl.loop(0, NDMA)
            def _drain(j):
                pltpu.make_async_copy(
                    tab_hbm.at[idx_v.at[pl.ds(0, IPD)]],
                    rows_v.at[pl.ds(0, IPD)], sem).wait()

            @pl.loop(0, G)
            def _interp(g):
                s0 = g * L
                xv = x0_v[pl.ds(s0, L)]
                yv = x1_v[pl.ds(s0, L)]
                zv = x2_v[pl.ds(s0, L)]
                xw = xv - xv.astype(jnp.int32).astype(jnp.float32)
                yw = yv - yv.astype(jnp.int32).astype(jnp.float32)
                zw = zv - zv.astype(jnp.int32).astype(jnp.float32)
                xw1 = 1.0 - xw
                yw1 = 1.0 - yw
                zw1 = 1.0 - zw
                for f in range(FEATURE_DIM):
                    fb = f * 8 * N
                    v = [rows_v[pl.ds(fb + k * N + s0, L)] for k in range(8)]
                    c00 = v[0] * xw1 + v[1] * xw
                    c01 = v[4] * xw1 + v[5] * xw
                    c10 = v[3] * xw1 + v[2] * xw
                    c11 = v[7] * xw1 + v[6] * xw
                    c0 = c00 * yw1 + c10 * yw
                    c1 = c01 * yw1 + c11 * yw
                    outval = c0 * zw1 + c1 * zw
                    fcol = 2 * level + f
                    off = ((fcol // 8) * (8 * N) + ((s0 >> 7) << 10)
                           + (fcol % 8) * 128 + (s0 & 127))
                    out_v[pl.ds(off, L)] = outval

        for r in range(4):
            pltpu.sync_copy(
                out_v.at[pl.ds(r * 8 * N, 8 * N)],
                out_hbm.at[pl.ds(r * (BATCH * 8) + gbase * 8, 8 * N)])


@jax.jit
def kernel(x, tables):
    b = jnp.exp((jnp.log(MAX_RES) - jnp.log(MIN_RES)) / (NUM_LEVELS - 1))
    res_list = [jnp.floor(MIN_RES * b ** level) for level in range(NUM_LEVELS)]
    res_bcast = jnp.broadcast_to(
        jnp.stack(res_list).reshape(NUM_LEVELS, 1), (NUM_LEVELS, L)).reshape(-1)

    x0 = x[:, 0]
    x1 = x[:, 1]
    x2 = x[:, 2]
    tab_flat = tables.reshape(NUM_LEVELS, TABLE_SZ // 128, 128, FEATURE_DIM)
    tab_flat = tab_flat.transpose(0, 1, 3, 2).reshape(-1)

    run = pl.kernel(
        _body,
        out_type=jax.ShapeDtypeStruct((BATCH * OUT_DIM,), jnp.float32),
        mesh=plsc.VectorSubcoreMesh(core_axis_name="c", subcore_axis_name="s"),
        scratch_types=[
            pltpu.VMEM((N,), jnp.float32),
            pltpu.VMEM((N,), jnp.float32),
            pltpu.VMEM((N,), jnp.float32),
            pltpu.VMEM((NUM_LEVELS * L,), jnp.float32),
            pltpu.VMEM((NIDX,), jnp.int32),
            pltpu.VMEM((NIDX,), jnp.float32),
            pltpu.VMEM((N * OUT_DIM,), jnp.float32),
            pltpu.SemaphoreType.DMA,
        ],
        compiler_params=pltpu.CompilerParams(
            needs_layout_passes=False, use_tc_tiling_on_sc=False),
    )
    out_flat = run(x0, x1, x2, tab_flat, res_bcast)
    out4 = out_flat.reshape(4, BATCH // 128, 8, 128)
    return out4.transpose(1, 3, 0, 2).reshape(BATCH, OUT_DIM)


# speedup vs baseline: 106.2502x; 1.1781x over previous
import jax
import jax.numpy as jnp
import numpy as np
from jax import lax
from jax.experimental import pallas as pl
from jax.experimental.pallas import tpu as pltpu
from jax.experimental.pallas import tpu_sc as plsc

NUM_LEVELS = 16
MIN_RES = 16.0
MAX_RES = 512.0
TABLE_SZ = 524288
MASK = TABLE_SZ - 1
FEATURE_DIM = 2
BATCH = 262144
C1 = np.int32(-1640531535)
C2 = np.int32(805459861)

NC, NS, L = 2, 16, 16
NW = NC * NS
SAMP = BATCH // NW
N = 1024
NCHUNK = SAMP // N
G = N // L
NROW = 8 * N
IPD = 256
NDMA = NROW // IPD
OUT_DIM = 2 * NUM_LEVELS

NBLK = NUM_LEVELS * TABLE_SZ // 128
BPT = NBLK // NS
BB = 16
NBATCH = BPT // BB


def _body(x0_hbm, x1_hbm, x2_hbm, tab_hbm, res_hbm, out_hbm, pair_hbm,
          x0_v, x1_v, x2_v, res_v, idx_v, rows_v, out_v,
          st_in, st_out, sem):
    sid = lax.axis_index("s")
    wid = sid * NC + lax.axis_index("c")
    base = wid * SAMP

    pltpu.sync_copy(res_hbm, res_v)

    iota = lax.iota(jnp.int32, L)

    zcol = jnp.full((L,), 0, jnp.int32)
    onecol = jnp.full((L,), 1, jnp.int32)

    @pl.loop(0, NBATCH)
    def _relayout(t):
        blk0 = sid * BPT + t * BB
        pltpu.sync_copy(tab_hbm.at[pl.ds(blk0 * 256, BB * 256)], st_in)
        for bk in range(BB):
            for g in range(8):
                a = st_in[pl.ds(bk * 256 + g * 16, L)]
                b = st_in[pl.ds(bk * 256 + 128 + g * 16, L)]
                rowids = bk * 128 + g * 16 + iota
                plsc.store_scatter(st_out, [rowids, zcol], a)
                plsc.store_scatter(st_out, [rowids, onecol], b)
        pltpu.sync_copy(st_out, pair_hbm.at[pl.ds(blk0 * 128, BB * 128)])

    plsc.subcore_barrier()

    pair2d = pair_hbm

    @pl.loop(0, NCHUNK)
    def _chunk(c):
        gbase = base + c * N
        pltpu.sync_copy(x0_hbm.at[pl.ds(gbase, N)], x0_v)
        pltpu.sync_copy(x1_hbm.at[pl.ds(gbase, N)], x1_v)
        pltpu.sync_copy(x2_hbm.at[pl.ds(gbase, N)], x2_v)

        @pl.loop(0, NUM_LEVELS)
        def _level(level):
            res = res_v[pl.ds(level * L, L)]
            lbase = level * jnp.full((L,), TABLE_SZ, jnp.int32)

            @pl.loop(0, G)
            def _hash(g):
                s0 = g * L
                xv = x0_v[pl.ds(s0, L)]
                yv = x1_v[pl.ds(s0, L)]
                zv = x2_v[pl.ds(s0, L)]
                sx = res * xv
                sy = res * yv
                sz = res * zv
                xl = sx.astype(jnp.int32)
                yl = sy.astype(jnp.int32)
                zl = sz.astype(jnp.int32)
                xh = jnp.where(sx > xl.astype(jnp.float32), xl + 1, xl)
                yh = jnp.where(sy > yl.astype(jnp.float32), yl + 1, yl)
                zh = jnp.where(sz > zl.astype(jnp.float32), zl + 1, zl)
                cyl = C1 * yl
                cyh = C1 * yh
                czl = C2 * zl
                czh = C2 * zh
                a = cyl ^ czl
                b = cyh ^ czl
                cc = cyl ^ czh
                d = cyh ^ czh
                hs = (xl ^ a, xh ^ a, xh ^ b, xl ^ b,
                      xl ^ cc, xh ^ cc, xh ^ d, xl ^ d)
                for k in range(8):
                    idx_v[pl.ds(k * N + s0, L)] = (hs[k] & MASK) + lbase

            @pl.loop(0, NDMA)
            def _fire(j):
                pltpu.async_copy(
                    pair2d.at[idx_v.at[pl.ds(j * IPD, IPD)]],
                    rows_v.at[pl.ds(j * IPD, IPD)], sem)

            @pl.loop(0, NDMA)
            def _drain(j):
                pltpu.make_async_copy(
                    pair2d.at[idx_v.at[pl.ds(0, IPD)]],
                    rows_v.at[pl.ds(0, IPD)], sem).wait()

            @pl.loop(0, G)
            def _interp(g):
                s0 = g * L
                xv = x0_v[pl.ds(s0, L)]
                yv = x1_v[pl.ds(s0, L)]
                zv = x2_v[pl.ds(s0, L)]
                xw = xv - xv.astype(jnp.int32).astype(jnp.float32)
                yw = yv - yv.astype(jnp.int32).astype(jnp.float32)
                zw = zv - zv.astype(jnp.int32).astype(jnp.float32)
                xw1 = 1.0 - xw
                yw1 = 1.0 - yw
                zw1 = 1.0 - zw
                srow = s0 + iota
                for f in range(FEATURE_DIM):
                    col = jnp.full((L,), f, jnp.int32)
                    v = [plsc.load_gather(rows_v, [k * N + srow, col])
                         for k in range(8)]
                    c00 = v[0] * xw1 + v[1] * xw
                    c01 = v[4] * xw1 + v[5] * xw
                    c10 = v[3] * xw1 + v[2] * xw
                    c11 = v[7] * xw1 + v[6] * xw
                    c0 = c00 * yw1 + c10 * yw
                    c1 = c01 * yw1 + c11 * yw
                    outval = c0 * zw1 + c1 * zw
                    fcol = 2 * level + f
                    off = ((fcol // 8) * (8 * N) + ((s0 >> 7) << 10)
                           + (fcol % 8) * 128 + (s0 & 127))
                    out_v[pl.ds(off, L)] = outval

        for r in range(4):
            pltpu.sync_copy(
                out_v.at[pl.ds(r * 8 * N, 8 * N)],
                out_hbm.at[pl.ds(r * (BATCH * 8) + gbase * 8, 8 * N)])


@jax.jit
def kernel(x, tables):
    b = jnp.exp((jnp.log(MAX_RES) - jnp.log(MIN_RES)) / (NUM_LEVELS - 1))
    res_list = [jnp.floor(MIN_RES * b ** level) for level in range(NUM_LEVELS)]
    res_bcast = jnp.broadcast_to(
        jnp.stack(res_list).reshape(NUM_LEVELS, 1), (NUM_LEVELS, L)).reshape(-1)

    x0 = x[:, 0]
    x1 = x[:, 1]
    x2 = x[:, 2]
    tab_flat = tables.reshape(NUM_LEVELS, TABLE_SZ // 128, 128, FEATURE_DIM)
    tab_flat = tab_flat.transpose(0, 1, 3, 2).reshape(-1)

    run = pl.kernel(
        _body,
        out_type=(
            jax.ShapeDtypeStruct((BATCH * OUT_DIM,), jnp.float32),
            jax.ShapeDtypeStruct((NUM_LEVELS * TABLE_SZ, FEATURE_DIM),
                                 jnp.float32),
        ),
        mesh=plsc.VectorSubcoreMesh(core_axis_name="c", subcore_axis_name="s"),
        scratch_types=[
            pltpu.VMEM((N,), jnp.float32),
            pltpu.VMEM((N,), jnp.float32),
            pltpu.VMEM((N,), jnp.float32),
            pltpu.VMEM((NUM_LEVELS * L,), jnp.float32),
            pltpu.VMEM((NROW,), jnp.int32),
            pltpu.VMEM((NROW, FEATURE_DIM), jnp.float32),
            pltpu.VMEM((N * OUT_DIM,), jnp.float32),
            pltpu.VMEM((BB * 256,), jnp.float32),
            pltpu.VMEM((BB * 128, FEATURE_DIM), jnp.float32),
            pltpu.SemaphoreType.DMA,
        ],
        compiler_params=pltpu.CompilerParams(
            needs_layout_passes=False, use_tc_tiling_on_sc=False),
    )
    out_flat, _ = run(x0, x1, x2, tab_flat, res_bcast)
    out4 = out_flat.reshape(4, BATCH // 128, 8, 128)
    return out4.transpose(1, 3, 0, 2).reshape(BATCH, OUT_DIM)


# speedup vs baseline: 123.8012x; 1.1652x over previous
import jax
import jax.numpy as jnp
import numpy as np
from jax import lax
from jax.experimental import pallas as pl
from jax.experimental.pallas import tpu as pltpu
from jax.experimental.pallas import tpu_sc as plsc

NUM_LEVELS = 16
MIN_RES = 16.0
MAX_RES = 512.0
TABLE_SZ = 524288
MASK = TABLE_SZ - 1
FEATURE_DIM = 2
BATCH = 262144
C1 = np.int32(-1640531535)
C2 = np.int32(805459861)

NC, NS, L = 2, 16, 16
NW = NC * NS
SAMP = BATCH // NW
N = 512
NCHUNK = SAMP // N
G = N // L
NROW = 8 * N
IPD = 256
NDMA = NROW // IPD
OUT_DIM = 2 * NUM_LEVELS

NBLK = NUM_LEVELS * TABLE_SZ // 128
BPT = NBLK // NS
BB = 16
NBATCH = BPT // BB


def _body(x0_hbm, x1_hbm, x2_hbm, tab_hbm, res_hbm, out_hbm, pair_hbm,
          x0_v, x1_v, x2_v, res_v, idx_v, rows_v, out_v,
          st_in, st_out, sem):
    sid = lax.axis_index("s")
    wid = sid * NC + lax.axis_index("c")
    base = wid * SAMP

    pltpu.sync_copy(res_hbm, res_v)

    iota = lax.iota(jnp.int32, L)

    zcol = jnp.full((L,), 0, jnp.int32)
    onecol = jnp.full((L,), 1, jnp.int32)

    @pl.loop(0, NBATCH)
    def _relayout(t):
        blk0 = sid * BPT + t * BB
        pltpu.sync_copy(tab_hbm.at[pl.ds(blk0 * 256, BB * 256)], st_in)
        for bk in range(BB):
            for g in range(8):
                a = st_in[pl.ds(bk * 256 + g * 16, L)]
                b = st_in[pl.ds(bk * 256 + 128 + g * 16, L)]
                rowids = bk * 128 + g * 16 + iota
                plsc.store_scatter(st_out, [rowids, zcol], a)
                plsc.store_scatter(st_out, [rowids, onecol], b)
        pltpu.sync_copy(st_out, pair_hbm.at[pl.ds(blk0 * 128, BB * 128)])

    plsc.subcore_barrier()

    pair2d = pair_hbm

    def hash_into(level, ob):
        res = res_v[pl.ds(level * L, L)]
        lbase = level * jnp.full((L,), TABLE_SZ, jnp.int32)

        @pl.loop(0, G)
        def _hash(g):
            s0 = g * L
            xv = x0_v[pl.ds(s0, L)]
            yv = x1_v[pl.ds(s0, L)]
            zv = x2_v[pl.ds(s0, L)]
            sx = res * xv
            sy = res * yv
            sz = res * zv
            xl = sx.astype(jnp.int32)
            yl = sy.astype(jnp.int32)
            zl = sz.astype(jnp.int32)
            xh = jnp.where(sx > xl.astype(jnp.float32), xl + 1, xl)
            yh = jnp.where(sy > yl.astype(jnp.float32), yl + 1, yl)
            zh = jnp.where(sz > zl.astype(jnp.float32), zl + 1, zl)
            cyl = C1 * yl
            cyh = C1 * yh
            czl = C2 * zl
            czh = C2 * zh
            a = cyl ^ czl
            b = cyh ^ czl
            cc = cyl ^ czh
            d = cyh ^ czh
            hs = (xl ^ a, xh ^ a, xh ^ b, xl ^ b,
                  xl ^ cc, xh ^ cc, xh ^ d, xl ^ d)
            for k in range(8):
                idx_v[pl.ds(ob + k * N + s0, L)] = (hs[k] & MASK) + lbase

    def fire(ob):
        @pl.loop(0, NDMA)
        def _fire(j):
            pltpu.async_copy(
                pair2d.at[idx_v.at[pl.ds(ob + j * IPD, IPD)]],
                rows_v.at[pl.ds(ob + j * IPD, IPD)], sem)

    def drain():
        @pl.loop(0, NDMA)
        def _drain(j):
            pltpu.make_async_copy(
                pair2d.at[idx_v.at[pl.ds(0, IPD)]],
                rows_v.at[pl.ds(0, IPD)], sem).wait()

    def interp(level, ob):
        @pl.loop(0, G)
        def _interp(g):
            s0 = g * L
            xv = x0_v[pl.ds(s0, L)]
            yv = x1_v[pl.ds(s0, L)]
            zv = x2_v[pl.ds(s0, L)]
            xw = xv - xv.astype(jnp.int32).astype(jnp.float32)
            yw = yv - yv.astype(jnp.int32).astype(jnp.float32)
            zw = zv - zv.astype(jnp.int32).astype(jnp.float32)
            xw1 = 1.0 - xw
            yw1 = 1.0 - yw
            zw1 = 1.0 - zw
            srow = s0 + iota
            for f in range(FEATURE_DIM):
                col = jnp.full((L,), f, jnp.int32)
                v = [plsc.load_gather(rows_v, [ob + k * N + srow, col])
                     for k in range(8)]
                c00 = v[0] * xw1 + v[1] * xw
                c01 = v[4] * xw1 + v[5] * xw
                c10 = v[3] * xw1 + v[2] * xw
                c11 = v[7] * xw1 + v[6] * xw
                c0 = c00 * yw1 + c10 * yw
                c1 = c01 * yw1 + c11 * yw
                outval = c0 * zw1 + c1 * zw
                fcol = 2 * level + f
                off = ((fcol // 8) * (8 * N) + ((s0 >> 7) << 10)
                       + (fcol % 8) * 128 + (s0 & 127))
                out_v[pl.ds(off, L)] = outval

    @pl.loop(0, NCHUNK)
    def _chunk(c):
        gbase = base + c * N
        pltpu.sync_copy(x0_hbm.at[pl.ds(gbase, N)], x0_v)
        pltpu.sync_copy(x1_hbm.at[pl.ds(gbase, N)], x1_v)
        pltpu.sync_copy(x2_hbm.at[pl.ds(gbase, N)], x2_v)

        hash_into(0, 0)
        fire(0)

        @pl.loop(0, NUM_LEVELS)
        def _level(level):
            ob = (level & 1) * NROW
            nob = NROW - ob

            @pl.when(level < NUM_LEVELS - 1)
            def _():
                hash_into(level + 1, nob)

            drain()

            @pl.when(level < NUM_LEVELS - 1)
            def _():
                fire(nob)

            interp(level, ob)

        for r in range(4):
            pltpu.sync_copy(
                out_v.at[pl.ds(r * 8 * N, 8 * N)],
                out_hbm.at[pl.ds(r * (BATCH * 8) + gbase * 8, 8 * N)])


@jax.jit
def kernel(x, tables):
    b = jnp.exp((jnp.log(MAX_RES) - jnp.log(MIN_RES)) / (NUM_LEVELS - 1))
    res_list = [jnp.floor(MIN_RES * b ** level) for level in range(NUM_LEVELS)]
    res_bcast = jnp.broadcast_to(
        jnp.stack(res_list).reshape(NUM_LEVELS, 1), (NUM_LEVELS, L)).reshape(-1)

    x0 = x[:, 0]
    x1 = x[:, 1]
    x2 = x[:, 2]
    tab_flat = tables.reshape(NUM_LEVELS, TABLE_SZ // 128, 128, FEATURE_DIM)
    tab_flat = tab_flat.transpose(0, 1, 3, 2).reshape(-1)

    run = pl.kernel(
        _body,
        out_type=(
            jax.ShapeDtypeStruct((BATCH * OUT_DIM,), jnp.float32),
            jax.ShapeDtypeStruct((NUM_LEVELS * TABLE_SZ, FEATURE_DIM),
                                 jnp.float32),
        ),
        mesh=plsc.VectorSubcoreMesh(core_axis_name="c", subcore_axis_name="s"),
        scratch_types=[
            pltpu.VMEM((N,), jnp.float32),
            pltpu.VMEM((N,), jnp.float32),
            pltpu.VMEM((N,), jnp.float32),
            pltpu.VMEM((NUM_LEVELS * L,), jnp.float32),
            pltpu.VMEM((2 * NROW,), jnp.int32),
            pltpu.VMEM((2 * NROW, FEATURE_DIM), jnp.float32),
            pltpu.VMEM((N * OUT_DIM,), jnp.float32),
            pltpu.VMEM((BB * 256,), jnp.float32),
            pltpu.VMEM((BB * 128, FEATURE_DIM), jnp.float32),
            pltpu.SemaphoreType.DMA,
        ],
        compiler_params=pltpu.CompilerParams(
            needs_layout_passes=False, use_tc_tiling_on_sc=False),
    )
    out_flat, _ = run(x0, x1, x2, tab_flat, res_bcast)
    out4 = out_flat.reshape(4, BATCH // 128, 8, 128)
    return out4.transpose(1, 3, 0, 2).reshape(BATCH, OUT_DIM)
